# baseline (speedup 1.0000x reference)
import functools

import jax
import jax.numpy as jnp
from jax import lax
from jax.experimental import pallas as pl
from jax.experimental.pallas import tpu as pltpu
from jax.experimental.pallas import tpu_sc as plsc

LENGTH = 1000000
BATCH = 1024
NB = 977
NOUT = NB * BATCH
PAD = NOUT - LENGTH

W = 32
CHUNK = 31744
ROWS = 248
VREGS = CHUNK // 16
N2 = W * CHUNK
KROWS = N2 // 128
KBLK = KROWS // 8
NBKT = 4096
SENT = 1 << 23


def _keys_body(kd_ref, out_ref):
    pid = pl.program_id(0)
    k0 = kd_ref[0]
    k1 = kd_ref[1]
    k2 = k0 ^ k1 ^ jnp.uint32(0x1BD11BDA)
    r = lax.broadcasted_iota(jnp.uint32, (KBLK, 128), 0)
    c = lax.broadcasted_iota(jnp.uint32, (KBLK, 128), 1)
    i = (r + jnp.uint32(KBLK) * pid.astype(jnp.uint32)) * jnp.uint32(128) + c
    ks = (k0, k1, k2)
    x0 = jnp.zeros_like(i) + k0
    x1 = i + k1
    rots = ((13, 15, 26, 6), (17, 29, 16, 24))
    for rnd in range(5):
        for rot in rots[rnd % 2]:
            x0 = x0 + x1
            x1 = (x1 << jnp.uint32(rot)) | (x1 >> jnp.uint32(32 - rot))
            x1 = x0 ^ x1
        x0 = x0 + ks[(rnd + 1) % 3]
        x1 = x1 + ks[(rnd + 2) % 3] + jnp.uint32(rnd + 1)
    bits = x0 ^ x1
    key23 = (bits >> jnp.uint32(9)).astype(jnp.int32)
    out_ref[...] = jnp.where(i.astype(jnp.int32) < LENGTH, key23, SENT)


def _wid():
    return lax.axis_index("c") * 16 + lax.axis_index("s")


def _hist_body(shift, keys_hbm, h_hbm, kv, h):
    w = _wid()

    def zero(i, _):
        h[pl.ds(i * 16, 16)] = jnp.zeros((16,), jnp.int32)
        return 0

    lax.fori_loop(0, NBKT // 16, zero, 0)
    pltpu.sync_copy(keys_hbm.at[pl.ds(w * CHUNK, CHUNK)], kv)

    def body(i, _):
        k = kv[pl.ds(i * 16, 16)]
        d = (k >> shift) & 0xFFF
        c1, l1 = plsc.scan_count(d)
        plsc.addupdate_scatter(h, [d], c1, mask=l1)
        return 0

    lax.fori_loop(0, VREGS, body, 0)
    pltpu.sync_copy(h, h_hbm.at[w])


def _scan_body(h_hbm, b_hbm, rowv, acc):
    c = lax.axis_index("c")
    s = lax.axis_index("s")

    @pl.when(jnp.logical_and(c == 0, s == 0))
    def _():
        def zeroa(i, _):
            acc[pl.ds(i * 16, 16)] = jnp.zeros((16,), jnp.int32)
            return 0

        lax.fori_loop(0, NBKT // 16, zeroa, 0)

        def addrow(wp, _):
            pltpu.sync_copy(h_hbm.at[wp], rowv)

            def addv(i, _):
                sl = pl.ds(i * 16, 16)
                acc[sl] = acc[sl] + rowv[sl]
                return 0

            lax.fori_loop(0, NBKT // 16, addv, 0)
            return 0

        lax.fori_loop(0, W, addrow, 0)

        def scan(i, carry):
            sl = pl.ds(i * 16, 16)
            v = acc[sl]
            s_incl = plsc.cumsum(v)
            acc[sl] = s_incl - v + carry
            return carry + jnp.sum(v)

        lax.fori_loop(0, NBKT // 16, scan, jnp.int32(0))

        def rowloop(wp, _):
            pltpu.sync_copy(acc, b_hbm.at[wp])
            pltpu.sync_copy(h_hbm.at[wp], rowv)

            def addv(i, _):
                sl = pl.ds(i * 16, 16)
                acc[sl] = acc[sl] + rowv[sl]
                return 0

            lax.fori_loop(0, NBKT // 16, addv, 0)
            return 0

        lax.fori_loop(0, W, rowloop, 0)


def _perm1_body(keys_hbm, b1_hbm, p2_hbm, kv, occ, destb, payb, sem):
    w = _wid()
    pltpu.sync_copy(keys_hbm.at[pl.ds(w * CHUNK, CHUNK)], kv)
    pltpu.sync_copy(b1_hbm.at[w], occ)

    def body(i, _):
        k = kv[pl.ds(i * 16, 16)]
        d = k & 0xFFF
        cnt, last = plsc.scan_count(d)
        old = plsc.load_gather(occ, [d])
        plsc.store_scatter(occ, [d], old + cnt, mask=last)
        destb[pl.ds(i * 16, 16)] = old + cnt - 1
        gid = w * CHUNK + i * 16 + lax.iota(jnp.int32, 16)
        idx1 = jnp.where(gid < LENGTH, gid + 1, 0)
        payb[pl.ds(i * 16, 16)] = ((k >> 12) << 20) | idx1
        return 0

    lax.fori_loop(0, VREGS, body, 0)
    pltpu.async_copy(payb, p2_hbm.at[destb], sem).wait()


def _perm2_body(p2_hbm, b2_hbm, out_hbm, kv, occ, destb, payb, sem):
    w = _wid()
    pltpu.sync_copy(p2_hbm.at[pl.ds(w * CHUNK, CHUNK)], kv)
    pltpu.sync_copy(b2_hbm.at[w], occ)

    def body(i, _):
        k = kv[pl.ds(i * 16, 16)]
        d = k >> 20
        cnt, last = plsc.scan_count(d)
        old = plsc.load_gather(occ, [d])
        plsc.store_scatter(occ, [d], old + cnt, mask=last)
        destb[pl.ds(i * 16, 16)] = old + cnt - 1
        payb[pl.ds(i * 16, 16)] = (k & 0xFFFFF) - 1
        return 0

    lax.fori_loop(0, VREGS, body, 0)
    pltpu.async_copy(payb, out_hbm.at[destb], sem).wait()


def kernel(key):
    kd = jax.random.key_data(key).astype(jnp.uint32)

    keys2d = pl.pallas_call(
        _keys_body,
        grid=(8,),
        in_specs=[pl.BlockSpec(memory_space=pltpu.SMEM)],
        out_specs=pl.BlockSpec((KBLK, 128), lambda i: (i, 0)),
        out_shape=jax.ShapeDtypeStruct((KROWS, 128), jnp.int32),
    )(kd)
    keys = keys2d.reshape(N2)

    mesh = plsc.VectorSubcoreMesh(core_axis_name="c", subcore_axis_name="s")
    cparams = pltpu.CompilerParams(needs_layout_passes=False)

    def make_hist(shift):
        return pl.kernel(
            functools.partial(_hist_body, shift),
            out_type=jax.ShapeDtypeStruct((W, NBKT), jnp.int32),
            mesh=mesh,
            compiler_params=cparams,
            scratch_types=[
                pltpu.VMEM((CHUNK,), jnp.int32),
                pltpu.VMEM((NBKT,), jnp.int32),
            ],
        )

    scan = pl.kernel(
        _scan_body,
        out_type=jax.ShapeDtypeStruct((W, NBKT), jnp.int32),
        mesh=mesh,
        compiler_params=cparams,
        scratch_types=[
            pltpu.VMEM((NBKT,), jnp.int32),
            pltpu.VMEM((NBKT,), jnp.int32),
        ],
    )

    h1 = make_hist(0)(keys)
    b1 = scan(h1)

    perm1 = pl.kernel(
        _perm1_body,
        out_type=jax.ShapeDtypeStruct((N2,), jnp.int32),
        mesh=mesh,
        compiler_params=cparams,
        scratch_types=[
            pltpu.VMEM((CHUNK,), jnp.int32),
            pltpu.VMEM((NBKT,), jnp.int32),
            pltpu.VMEM((CHUNK,), jnp.int32),
            pltpu.VMEM((CHUNK,), jnp.int32),
            pltpu.SemaphoreType.DMA,
        ],
    )
    p2 = perm1(keys, b1)

    h2 = make_hist(20)(p2)
    b2 = scan(h2)

    perm2 = pl.kernel(
        _perm2_body,
        out_type=jax.ShapeDtypeStruct((N2,), jnp.int32),
        mesh=mesh,
        compiler_params=cparams,
        scratch_types=[
            pltpu.VMEM((CHUNK,), jnp.int32),
            pltpu.VMEM((NBKT,), jnp.int32),
            pltpu.VMEM((CHUNK,), jnp.int32),
            pltpu.VMEM((CHUNK,), jnp.int32),
            pltpu.SemaphoreType.DMA,
        ],
    )
    flat = perm2(p2, b2)

    idxes = flat[:NOUT].reshape(NB, BATCH)
    pad_mask = jnp.zeros((NB, BATCH), jnp.bool_).at[NB - 1, BATCH - PAD:].set(True)
    return (idxes, pad_mask)


# speedup vs baseline: 1.0023x; 1.0023x over previous
import functools

import jax
import jax.numpy as jnp
from jax import lax
from jax.experimental import pallas as pl
from jax.experimental.pallas import tpu as pltpu
from jax.experimental.pallas import tpu_sc as plsc

LENGTH = 1000000
BATCH = 1024
NB = 977
NOUT = NB * BATCH
PAD = NOUT - LENGTH

W = 32
CHUNK = 31744
ROWS = 248
VREGS = CHUNK // 16
N2 = W * CHUNK
KROWS = N2 // 128
KBLK = KROWS // 8
NBKT = 4096
SENT = 1 << 23


def _keys_body(kd_ref, out_ref):
    pid = pl.program_id(0)
    k0 = kd_ref[0]
    k1 = kd_ref[1]
    k2 = k0 ^ k1 ^ jnp.uint32(0x1BD11BDA)
    r = lax.broadcasted_iota(jnp.uint32, (KBLK, 128), 0)
    c = lax.broadcasted_iota(jnp.uint32, (KBLK, 128), 1)
    i = (r + jnp.uint32(KBLK) * pid.astype(jnp.uint32)) * jnp.uint32(128) + c
    ks = (k0, k1, k2)
    x0 = jnp.zeros_like(i) + k0
    x1 = i + k1
    rots = ((13, 15, 26, 6), (17, 29, 16, 24))
    for rnd in range(5):
        for rot in rots[rnd % 2]:
            x0 = x0 + x1
            x1 = (x1 << jnp.uint32(rot)) | (x1 >> jnp.uint32(32 - rot))
            x1 = x0 ^ x1
        x0 = x0 + ks[(rnd + 1) % 3]
        x1 = x1 + ks[(rnd + 2) % 3] + jnp.uint32(rnd + 1)
    bits = x0 ^ x1
    key23 = (bits >> jnp.uint32(9)).astype(jnp.int32)
    out_ref[...] = jnp.where(i.astype(jnp.int32) < LENGTH, key23, SENT)


def _wid():
    return lax.axis_index("c") * 16 + lax.axis_index("s")


def _hist_body(shift, keys_hbm, h_hbm, kv, h):
    w = _wid()

    def zero(i, _):
        h[pl.ds(i * 16, 16)] = jnp.zeros((16,), jnp.int32)
        return 0

    lax.fori_loop(0, NBKT // 16, zero, 0)
    pltpu.sync_copy(keys_hbm.at[pl.ds(w * CHUNK, CHUNK)], kv)

    def body(i, _):
        k = kv[pl.ds(i * 16, 16)]
        d = (k >> shift) & 0xFFF
        c1, l1 = plsc.scan_count(d)
        plsc.addupdate_scatter(h, [d], c1, mask=l1)
        return 0

    lax.fori_loop(0, VREGS, body, 0)
    pltpu.sync_copy(h, h_hbm.at[w])


def _scan_body(h_hbm, b_hbm, rowv, acc):
    c = lax.axis_index("c")
    s = lax.axis_index("s")

    @pl.when(jnp.logical_and(c == 0, s == 0))
    def _():
        def zeroa(i, _):
            acc[pl.ds(i * 16, 16)] = jnp.zeros((16,), jnp.int32)
            return 0

        lax.fori_loop(0, NBKT // 16, zeroa, 0)

        def addrow(wp, _):
            pltpu.sync_copy(h_hbm.at[wp], rowv)

            def addv(i, _):
                sl = pl.ds(i * 16, 16)
                acc[sl] = acc[sl] + rowv[sl]
                return 0

            lax.fori_loop(0, NBKT // 16, addv, 0)
            return 0

        lax.fori_loop(0, W, addrow, 0)

        def scan(i, carry):
            sl = pl.ds(i * 16, 16)
            v = acc[sl]
            s_incl = plsc.cumsum(v)
            acc[sl] = s_incl - v + carry
            return carry + jnp.sum(v)

        lax.fori_loop(0, NBKT // 16, scan, jnp.int32(0))

        def rowloop(wp, _):
            pltpu.sync_copy(acc, b_hbm.at[wp])
            pltpu.sync_copy(h_hbm.at[wp], rowv)

            def addv(i, _):
                sl = pl.ds(i * 16, 16)
                acc[sl] = acc[sl] + rowv[sl]
                return 0

            lax.fori_loop(0, NBKT // 16, addv, 0)
            return 0

        lax.fori_loop(0, W, rowloop, 0)


def _perm1_body(keys_hbm, b1_hbm, p2_hbm, kv, occ, destb, payb, sem):
    w = _wid()
    pltpu.sync_copy(keys_hbm.at[pl.ds(w * CHUNK, CHUNK)], kv)
    pltpu.sync_copy(b1_hbm.at[w], occ)

    def body(i, _):
        k = kv[pl.ds(i * 16, 16)]
        d = k & 0xFFF
        cnt, last = plsc.scan_count(d)
        old = plsc.load_gather(occ, [d])
        plsc.store_scatter(occ, [d], old + cnt, mask=last)
        destb[pl.ds(i * 16, 16)] = old + cnt - 1
        gid = w * CHUNK + i * 16 + lax.iota(jnp.int32, 16)
        idx1 = jnp.where(gid < LENGTH, gid + 1, 0)
        payb[pl.ds(i * 16, 16)] = ((k >> 12) << 20) | idx1
        return 0

    lax.fori_loop(0, VREGS, body, 0)
    pltpu.async_copy(payb, p2_hbm.at[destb], sem).wait()


def _perm2_body(p2_hbm, b2_hbm, out_hbm, kv, occ, destb, payb, sem):
    w = _wid()
    pltpu.sync_copy(p2_hbm.at[pl.ds(w * CHUNK, CHUNK)], kv)
    pltpu.sync_copy(b2_hbm.at[w], occ)

    def body(i, _):
        k = kv[pl.ds(i * 16, 16)]
        d = (k >> 20) & 0xFFF
        cnt, last = plsc.scan_count(d)
        old = plsc.load_gather(occ, [d])
        plsc.store_scatter(occ, [d], old + cnt, mask=last)
        destb[pl.ds(i * 16, 16)] = old + cnt - 1
        payb[pl.ds(i * 16, 16)] = (k & 0xFFFFF) - 1
        return 0

    lax.fori_loop(0, VREGS, body, 0)
    pltpu.async_copy(payb, out_hbm.at[destb], sem).wait()


def kernel(key):
    kd = jax.random.key_data(key).astype(jnp.uint32)

    keys2d = pl.pallas_call(
        _keys_body,
        grid=(8,),
        in_specs=[pl.BlockSpec(memory_space=pltpu.SMEM)],
        out_specs=pl.BlockSpec((KBLK, 128), lambda i: (i, 0)),
        out_shape=jax.ShapeDtypeStruct((KROWS, 128), jnp.int32),
    )(kd)
    keys = keys2d.reshape(N2)

    mesh = plsc.VectorSubcoreMesh(core_axis_name="c", subcore_axis_name="s")
    cparams = pltpu.CompilerParams(needs_layout_passes=False)

    def make_hist(shift):
        return pl.kernel(
            functools.partial(_hist_body, shift),
            out_type=jax.ShapeDtypeStruct((W, NBKT), jnp.int32),
            mesh=mesh,
            compiler_params=cparams,
            scratch_types=[
                pltpu.VMEM((CHUNK,), jnp.int32),
                pltpu.VMEM((NBKT,), jnp.int32),
            ],
        )

    scan = pl.kernel(
        _scan_body,
        out_type=jax.ShapeDtypeStruct((W, NBKT), jnp.int32),
        mesh=mesh,
        compiler_params=cparams,
        scratch_types=[
            pltpu.VMEM((NBKT,), jnp.int32),
            pltpu.VMEM((NBKT,), jnp.int32),
        ],
    )

    h1 = make_hist(0)(keys)
    b1 = scan(h1)

    perm1 = pl.kernel(
        _perm1_body,
        out_type=jax.ShapeDtypeStruct((N2,), jnp.int32),
        mesh=mesh,
        compiler_params=cparams,
        scratch_types=[
            pltpu.VMEM((CHUNK,), jnp.int32),
            pltpu.VMEM((NBKT,), jnp.int32),
            pltpu.VMEM((CHUNK,), jnp.int32),
            pltpu.VMEM((CHUNK,), jnp.int32),
            pltpu.SemaphoreType.DMA,
        ],
    )
    p2 = perm1(keys, b1)

    h2 = make_hist(20)(p2)
    b2 = scan(h2)

    perm2 = pl.kernel(
        _perm2_body,
        out_type=jax.ShapeDtypeStruct((N2,), jnp.int32),
        mesh=mesh,
        compiler_params=cparams,
        scratch_types=[
            pltpu.VMEM((CHUNK,), jnp.int32),
            pltpu.VMEM((NBKT,), jnp.int32),
            pltpu.VMEM((CHUNK,), jnp.int32),
            pltpu.VMEM((CHUNK,), jnp.int32),
            pltpu.SemaphoreType.DMA,
        ],
    )
    flat = perm2(p2, b2)

    idxes = flat[:NOUT].reshape(NB, BATCH)
    pad_mask = jnp.zeros((NB, BATCH), jnp.bool_).at[NB - 1, BATCH - PAD:].set(True)
    return (idxes, pad_mask)
